# baseline (speedup 1.0000x reference)
import functools

import jax
import jax.numpy as jnp
from jax import lax
from jax.experimental import pallas as pl
from jax.experimental.pallas import tpu as pltpu
from jax.experimental.pallas import tpu_sc as plsc

_PH, _PW = 7, 7
_SMIN, _SMAX = 12, 29
_NSZ = _SMAX - _SMIN + 1
_CROP = 32
_N = 1000
_NPAD = 1024
_H = _W = 64
_C = 128
_G = 40


def _resize_table():
    mats = []
    for s in range(_SMIN, _SMAX + 1):
        eye = jnp.eye(s, dtype=jnp.float32)
        r = jax.image.resize(eye, (_PH, s), method="bilinear")
        r = jnp.pad(r, ((0, 8 - _PH), (0, _CROP - s)))
        mats.append(r)
    return jnp.stack(mats)


_KY = 40


def _x_mask_table(rtab):
    eye = jnp.eye(8, dtype=jnp.float32)[:_PH]
    t = jnp.einsum("kqx,pe->kpqxe", rtab[:, :, :_SMAX], eye)
    return t.reshape(_NSZ, _PH, 8, _SMAX * 8)


def _global_y_table(rtab):
    t = jnp.zeros((_NSZ, 8, 8, _KY), jnp.float32)
    for off in range(8):
        t = t.at[:, off, :, off:off + _CROP].set(rtab)
    return t.reshape(_NSZ * 8, 8, _KY)


_INFO = plsc.get_sparse_core_info()
_NCORE, _NSUB = _INFO.num_cores, _INFO.num_subcores
_NWORK = _NCORE * _NSUB
_PER_W = _NPAD // _NWORK


def _sc_coords_kernel(rois_hbm, out_hbm, rv, ov):
    wid = lax.axis_index("s") * _NCORE + lax.axis_index("c")
    base = wid * _PER_W
    for f in range(4):
        pltpu.sync_copy(rois_hbm.at[pl.ds(f * _NPAD + base, _PER_W)],
                        rv.at[f])
    for h in range(_PER_W // 16):
        sl = pl.ds(16 * h, 16)
        x = rv[0, sl] * float(_W)
        y = rv[1, sl] * float(_H)
        w = rv[2, sl] * float(_W)
        hh = rv[3, sl] * float(_H)
        x1 = x.astype(jnp.int32)
        y1 = y.astype(jnp.int32)
        sx = x + w
        bbx = sx - x
        errx = (x - (sx - bbx)) + (w - bbx)
        fx = sx.astype(jnp.int32)
        x2 = fx - jnp.where((sx == fx.astype(jnp.float32)) & (errx < 0.0),
                            1, 0).astype(jnp.int32)
        sy = y + hh
        bby = sy - y
        erry = (y - (sy - bby)) + (hh - bby)
        fy = sy.astype(jnp.int32)
        y2 = fy - jnp.where((sy == fy.astype(jnp.float32)) & (erry < 0.0),
                            1, 0).astype(jnp.int32)
        kx = x2 - x1 - _SMIN
        y1a = y1 & ~7
        kyg = (y2 - y1 - _SMIN) * 8 + (y1 - y1a)
        ov[0, sl] = x1
        ov[1, sl] = kyg
        ov[2, sl] = kx
        ov[3, sl] = y1a
    for f in range(4):
        pltpu.sync_copy(ov.at[f],
                        out_hbm.at[pl.ds(f * _NPAD + base, _PER_W)])


def _sc_coords(rois):
    roisf = jnp.pad(rois, ((0, _NPAD - _N), (0, 0))).T.reshape(-1)
    mesh = plsc.VectorSubcoreMesh(core_axis_name="c", subcore_axis_name="s")
    k = functools.partial(
        pl.kernel,
        mesh=mesh,
        out_type=jax.ShapeDtypeStruct((4 * _NPAD,), jnp.int32),
        scratch_types=[
            pltpu.VMEM((4, _PER_W), jnp.float32),
            pltpu.VMEM((4, _PER_W), jnp.int32),
        ],
    )(_sc_coords_kernel)
    return k(roisf).reshape(4, _NPAD)


def _roi_kernel(coord_s, fm2_ref, wytab_ref, rxbig_ref, out_ref):
    i = pl.program_id(0)
    for g in range(_G):
        r = i * _G + g
        x1 = coord_s[0, r]
        kyg = coord_s[1, r]
        kx = coord_s[2, r]
        y1a = pl.multiple_of(coord_s[3, r], 8)

        wy = wytab_ref[kyg]
        rhs = fm2_ref[pl.ds(y1a, _KY), pl.ds(x1 * _C, _SMAX * _C)]

        chunks = [
            jax.lax.dot_general(
                wy, rhs[:, x * _C:(x + 1) * _C], (((1,), (0,)), ((), ())),
                preferred_element_type=jnp.float32,
            )
            for x in range(_SMAX)
        ]
        astk = jnp.concatenate(chunks, axis=0).astype(jnp.bfloat16)
        for p in range(_PH):
            op = jax.lax.dot_general(
                rxbig_ref[kx, p], astk, (((1,), (0,)), ((), ())),
                preferred_element_type=jnp.float32,
            )
            out_ref[g, p] = op[:_PW]


def kernel(feature_map, rois):
    coords = _sc_coords(rois)
    fm2 = feature_map[0].reshape(_H, _W * _C).astype(jnp.bfloat16)
    rxbig = _x_mask_table(_resize_table()).astype(jnp.bfloat16)
    wytab = _global_y_table(_resize_table()).astype(jnp.bfloat16)
    grid_spec = pltpu.PrefetchScalarGridSpec(
        num_scalar_prefetch=1,
        grid=(_N // _G,),
        in_specs=[
            pl.BlockSpec((_H, _W * _C), lambda i, s: (0, 0)),
            pl.BlockSpec((_NSZ * 8, 8, _KY), lambda i, s: (0, 0, 0)),
            pl.BlockSpec((_NSZ, _PH, 8, _SMAX * 8),
                         lambda i, s: (0, 0, 0, 0)),
        ],
        out_specs=pl.BlockSpec((_G, _PH, _PW, _C), lambda i, s: (i, 0, 0, 0)),
    )
    out = pl.pallas_call(
        _roi_kernel,
        grid_spec=grid_spec,
        out_shape=jax.ShapeDtypeStruct((_N, _PH, _PW, _C), jnp.float32),
    )(coords, fm2, wytab, rxbig)
    return out


# speedup vs baseline: 1.0840x; 1.0840x over previous
import functools

import jax
import jax.numpy as jnp
from jax import lax
from jax.experimental import pallas as pl
from jax.experimental.pallas import tpu as pltpu
from jax.experimental.pallas import tpu_sc as plsc

_PH, _PW = 7, 7
_SMIN, _SMAX = 12, 29
_NSZ = _SMAX - _SMIN + 1
_CROP = 32
_N = 1000
_NPAD = 1024
_H = _W = 64
_C = 128
_G = 40


def _resize_table():
    mats = []
    for s in range(_SMIN, _SMAX + 1):
        eye = jnp.eye(s, dtype=jnp.float32)
        r = jax.image.resize(eye, (_PH, s), method="bilinear")
        r = jnp.pad(r, ((0, 8 - _PH), (0, _CROP - s)))
        mats.append(r)
    return jnp.stack(mats)


_KY = 40


def _x_mask_table(rtab):
    eye = jnp.eye(8, dtype=jnp.float32)[:_PH]
    t = jnp.einsum("kqx,pe->kpqxe", rtab[:, :, :_SMAX], eye)
    return t.reshape(_NSZ, _PH, 8, _SMAX * 8)


def _global_y_table(rtab):
    t = jnp.zeros((_NSZ, 8, 8, _KY), jnp.float32)
    for off in range(8):
        t = t.at[:, off, :, off:off + _CROP].set(rtab)
    return t.reshape(_NSZ * 8, 8, _KY)


_INFO = plsc.get_sparse_core_info()
_NCORE, _NSUB = _INFO.num_cores, _INFO.num_subcores
_NWORK = _NCORE * _NSUB
_PER_W = _NPAD // _NWORK


def _sc_coords_kernel(rois_hbm, out_hbm, rv, ov):
    wid = lax.axis_index("s") * _NCORE + lax.axis_index("c")
    base = wid * _PER_W
    for f in range(4):
        pltpu.sync_copy(rois_hbm.at[pl.ds(f * _NPAD + base, _PER_W)],
                        rv.at[f])
    for h in range(_PER_W // 16):
        sl = pl.ds(16 * h, 16)
        x = rv[0, sl] * float(_W)
        y = rv[1, sl] * float(_H)
        w = rv[2, sl] * float(_W)
        hh = rv[3, sl] * float(_H)
        x1 = x.astype(jnp.int32)
        y1 = y.astype(jnp.int32)
        sx = x + w
        bbx = sx - x
        errx = (x - (sx - bbx)) + (w - bbx)
        fx = sx.astype(jnp.int32)
        x2 = fx - jnp.where((sx == fx.astype(jnp.float32)) & (errx < 0.0),
                            1, 0).astype(jnp.int32)
        sy = y + hh
        bby = sy - y
        erry = (y - (sy - bby)) + (hh - bby)
        fy = sy.astype(jnp.int32)
        y2 = fy - jnp.where((sy == fy.astype(jnp.float32)) & (erry < 0.0),
                            1, 0).astype(jnp.int32)
        kx = x2 - x1 - _SMIN
        y1a = y1 & ~7
        kyg = (y2 - y1 - _SMIN) * 8 + (y1 - y1a)
        ov[0, sl] = x1
        ov[1, sl] = kyg
        ov[2, sl] = kx
        ov[3, sl] = y1a
    for f in range(4):
        pltpu.sync_copy(ov.at[f],
                        out_hbm.at[pl.ds(f * _NPAD + base, _PER_W)])


def _sc_coords(rois):
    roisf = jnp.pad(rois, ((0, _NPAD - _N), (0, 0))).T.reshape(-1)
    mesh = plsc.VectorSubcoreMesh(core_axis_name="c", subcore_axis_name="s")
    k = functools.partial(
        pl.kernel,
        mesh=mesh,
        out_type=jax.ShapeDtypeStruct((4 * _NPAD,), jnp.int32),
        scratch_types=[
            pltpu.VMEM((4, _PER_W), jnp.float32),
            pltpu.VMEM((4, _PER_W), jnp.int32),
        ],
    )(_sc_coords_kernel)
    return k(roisf).reshape(4, _NPAD)


def _roi_kernel(coord_s, fm2_ref, wytab_ref, rxbig_ref, out_ref, stk_ref):
    i = pl.program_id(0)
    for g in range(_G):
        r = i * _G + g
        x1 = coord_s[0, r]
        kyg = coord_s[1, r]
        kx = coord_s[2, r]
        y1a = pl.multiple_of(coord_s[3, r], 8)

        wy = wytab_ref[kyg]
        rhs = fm2_ref[pl.ds(y1a, _KY), pl.ds(x1 * _C, _SMAX * _C)]

        a = jax.lax.dot_general(
            wy, rhs, (((1,), (0,)), ((), ())),
            preferred_element_type=jnp.float32,
        )
        ab = a.astype(jnp.bfloat16)
        for x in range(_SMAX):
            stk_ref[g, x * 8:(x + 1) * 8, :] = ab[:, x * _C:(x + 1) * _C]
        astk = stk_ref[g]
        for p in range(_PH):
            op = jax.lax.dot_general(
                rxbig_ref[kx, p], astk, (((1,), (0,)), ((), ())),
                preferred_element_type=jnp.float32,
            )
            out_ref[g, p] = op[:_PW]


def kernel(feature_map, rois):
    coords = _sc_coords(rois)
    fm2 = feature_map[0].reshape(_H, _W * _C).astype(jnp.bfloat16)
    rxbig = _x_mask_table(_resize_table()).astype(jnp.bfloat16)
    wytab = _global_y_table(_resize_table()).astype(jnp.bfloat16)
    grid_spec = pltpu.PrefetchScalarGridSpec(
        num_scalar_prefetch=1,
        grid=(_N // _G,),
        in_specs=[
            pl.BlockSpec((_H, _W * _C), lambda i, s: (0, 0)),
            pl.BlockSpec((_NSZ * 8, 8, _KY), lambda i, s: (0, 0, 0)),
            pl.BlockSpec((_NSZ, _PH, 8, _SMAX * 8),
                         lambda i, s: (0, 0, 0, 0)),
        ],
        out_specs=pl.BlockSpec((_G, _PH, _PW, _C), lambda i, s: (i, 0, 0, 0)),
        scratch_shapes=[pltpu.VMEM((_G, _SMAX * 8, _C), jnp.bfloat16)],
    )
    out = pl.pallas_call(
        _roi_kernel,
        grid_spec=grid_spec,
        out_shape=jax.ShapeDtypeStruct((_N, _PH, _PW, _C), jnp.float32),
    )(coords, fm2, wytab, rxbig)
    return out


# speedup vs baseline: 1.4421x; 1.3304x over previous
import functools

import jax
import jax.numpy as jnp
from jax import lax
from jax.experimental import pallas as pl
from jax.experimental.pallas import tpu as pltpu
from jax.experimental.pallas import tpu_sc as plsc

_PH, _PW = 7, 7
_SMIN, _SMAX = 12, 29
_NSZ = _SMAX - _SMIN + 1
_CROP = 32
_N = 1000
_NPAD = 1024
_H = _W = 64
_C = 128
_G = 50


def _resize_table():
    mats = []
    for s in range(_SMIN, _SMAX + 1):
        eye = jnp.eye(s, dtype=jnp.float32)
        r = jax.image.resize(eye, (_PH, s), method="bilinear")
        r = jnp.pad(r, ((0, 8 - _PH), (0, _CROP - s)))
        mats.append(r)
    return jnp.stack(mats)


_KY = 40


def _global_y_table(rtab):
    t = jnp.zeros((_NSZ, 8, 8, _KY), jnp.float32)
    for off in range(8):
        t = t.at[:, off, :, off:off + _CROP].set(rtab)
    return t.reshape(_NSZ * 8, 8, _KY)


_INFO = plsc.get_sparse_core_info()
_NCORE, _NSUB = _INFO.num_cores, _INFO.num_subcores
_NWORK = _NCORE * _NSUB
_PER_W = _NPAD // _NWORK


def _sc_coords_kernel(rois_hbm, out_hbm, rv, ov):
    wid = lax.axis_index("s") * _NCORE + lax.axis_index("c")
    base = wid * _PER_W
    for f in range(4):
        pltpu.sync_copy(rois_hbm.at[pl.ds(f * _NPAD + base, _PER_W)],
                        rv.at[f])
    for h in range(_PER_W // 16):
        sl = pl.ds(16 * h, 16)
        x = rv[0, sl] * float(_W)
        y = rv[1, sl] * float(_H)
        w = rv[2, sl] * float(_W)
        hh = rv[3, sl] * float(_H)
        x1 = x.astype(jnp.int32)
        y1 = y.astype(jnp.int32)
        sx = x + w
        bbx = sx - x
        errx = (x - (sx - bbx)) + (w - bbx)
        fx = sx.astype(jnp.int32)
        x2 = fx - jnp.where((sx == fx.astype(jnp.float32)) & (errx < 0.0),
                            1, 0).astype(jnp.int32)
        sy = y + hh
        bby = sy - y
        erry = (y - (sy - bby)) + (hh - bby)
        fy = sy.astype(jnp.int32)
        y2 = fy - jnp.where((sy == fy.astype(jnp.float32)) & (erry < 0.0),
                            1, 0).astype(jnp.int32)
        kx = x2 - x1 - _SMIN
        y1a = y1 & ~7
        kyg = (y2 - y1 - _SMIN) * 8 + (y1 - y1a)
        ov[0, sl] = x1
        ov[1, sl] = kyg
        ov[2, sl] = kx
        ov[3, sl] = y1a
    for f in range(4):
        pltpu.sync_copy(ov.at[f],
                        out_hbm.at[pl.ds(f * _NPAD + base, _PER_W)])


def _sc_coords(rois):
    roisf = jnp.pad(rois, ((0, _NPAD - _N), (0, 0))).T.reshape(-1)
    mesh = plsc.VectorSubcoreMesh(core_axis_name="c", subcore_axis_name="s")
    k = functools.partial(
        pl.kernel,
        mesh=mesh,
        out_type=jax.ShapeDtypeStruct((4 * _NPAD,), jnp.int32),
        scratch_types=[
            pltpu.VMEM((4, _PER_W), jnp.float32),
            pltpu.VMEM((4, _PER_W), jnp.int32),
        ],
    )(_sc_coords_kernel)
    return k(roisf).reshape(4, _NPAD)


def _roi_kernel(coord_s, fm2_ref, wytab_ref, rtab_ref, out_ref):
    i = pl.program_id(0)
    for g in range(_G):
        r = i * _G + g
        x1 = coord_s[0, r]
        kyg = coord_s[1, r]
        kx = coord_s[2, r]
        y1a = pl.multiple_of(coord_s[3, r], 8)

        wy = wytab_ref[kyg]
        rx = rtab_ref[kx]
        rhs = fm2_ref[pl.ds(y1a, _KY), pl.ds(x1 * _C, _SMAX * _C)]

        a = jax.lax.dot_general(
            wy, rhs, (((1,), (0,)), ((), ())),
            preferred_element_type=jnp.float32,
        )
        a3 = a.reshape(8, _SMAX, _C)
        for p in range(_PH):
            op = jax.lax.dot_general(
                rx[:, :_SMAX], a3[p], (((1,), (0,)), ((), ())),
                preferred_element_type=jnp.float32,
            )
            out_ref[g, p] = op[:_PW]


def kernel(feature_map, rois):
    coords = _sc_coords(rois)
    fm2 = feature_map[0].reshape(_H, _W * _C).astype(jnp.bfloat16)
    rtab = _resize_table()
    wytab = _global_y_table(_resize_table()).astype(jnp.bfloat16)
    grid_spec = pltpu.PrefetchScalarGridSpec(
        num_scalar_prefetch=1,
        grid=(_N // _G,),
        in_specs=[
            pl.BlockSpec((_H, _W * _C), lambda i, s: (0, 0)),
            pl.BlockSpec((_NSZ * 8, 8, _KY), lambda i, s: (0, 0, 0)),
            pl.BlockSpec((_NSZ, 8, _CROP), lambda i, s: (0, 0, 0)),
        ],
        out_specs=pl.BlockSpec((_G, _PH, _PW, _C), lambda i, s: (i, 0, 0, 0)),
    )
    out = pl.pallas_call(
        _roi_kernel,
        grid_spec=grid_spec,
        out_shape=jax.ShapeDtypeStruct((_N, _PH, _PW, _C), jnp.float32),
    )(coords, fm2, wytab, rtab)
    return out
